# baseline (speedup 1.0000x reference)
import functools

import jax
import jax.numpy as jnp
from jax import lax
from jax.experimental import pallas as pl
from jax.experimental.pallas import tpu as pltpu
from jax.experimental.pallas import tpu_sc as plsc

N = 10000
D = 128
E = 320000
NC, NS = 1, 16
NW = NC * NS
CHUNK = 128
NCHUNK = 160
EPT = NCHUNK * CHUNK
E_PAD = EPT * NW
NPAD = 10112
RPT = NPAD // NS


def _deg_body(edges_hbm, eye_hbm, zerosd_hbm, out_hbm,
              pair_v, e0_v, e1_v, acc, sem):
    c = lax.axis_index("c")
    s = lax.axis_index("s")
    wid = c * NS + s
    row0 = s * RPT
    pltpu.sync_copy(zerosd_hbm.at[pl.ds(row0, RPT)], acc.at[pl.ds(row0, RPT)])
    pltpu.sync_copy(eye_hbm.at[0], e0_v)
    pltpu.sync_copy(eye_hbm.at[1], e1_v)
    plsc.subcore_barrier()

    def body(i, carry):
        pltpu.async_copy(edges_hbm.at[wid, i], pair_v, sem).wait()
        pltpu.sync_copy(e0_v, acc.at[pair_v.at[0]], add=True)
        pltpu.sync_copy(e1_v, acc.at[pair_v.at[1]], add=True)
        return carry

    lax.fori_loop(0, NCHUNK, body, 0)
    plsc.subcore_barrier()
    pltpu.sync_copy(acc.at[pl.ds(row0, RPT)], out_hbm.at[c, pl.ds(row0, RPT)])


def _prop_body(t_hbm, edges_hbm, zerosd_hbm, out_hbm,
               idx_v, rows0_v, rows1_v, acc, sem_i, sem_g0, sem_g1):
    c = lax.axis_index("c")
    s = lax.axis_index("s")
    wid = c * NS + s
    row0 = s * RPT
    pltpu.sync_copy(zerosd_hbm.at[pl.ds(row0, RPT)], acc.at[pl.ds(row0, RPT)])
    plsc.subcore_barrier()

    rows_v = (rows0_v, rows1_v)
    sem_g = (sem_g0, sem_g1)

    def wait_idx():
        pltpu.make_async_copy(edges_hbm.at[wid, 0], idx_v.at[0], sem_i).wait()

    pltpu.async_copy(edges_hbm.at[wid, 0], idx_v.at[0], sem_i)
    pltpu.async_copy(edges_hbm.at[wid, 1], idx_v.at[1], sem_i)
    pltpu.async_copy(edges_hbm.at[wid, 2], idx_v.at[2], sem_i)
    wait_idx()
    pltpu.async_copy(t_hbm.at[idx_v.at[0, 0]], rows0_v, sem_g0)

    def body(g, carry):
        for p in range(4):
            i = 4 * g + p

            @pl.when(i + 1 < NCHUNK)
            def _():
                wait_idx()
                pltpu.async_copy(t_hbm.at[idx_v.at[(p + 1) % 4, 0]],
                                 rows_v[(p + 1) % 2], sem_g[(p + 1) % 2])

            @pl.when(i + 3 < NCHUNK)
            def _():
                pltpu.async_copy(edges_hbm.at[wid, i + 3],
                                 idx_v.at[(p + 3) % 4], sem_i)

            pltpu.make_async_copy(t_hbm.at[idx_v.at[p, 0]],
                                  rows_v[p % 2], sem_g[p % 2]).wait()
            pltpu.sync_copy(rows_v[p % 2], acc.at[idx_v.at[p, 1]], add=True)
        return carry

    lax.fori_loop(0, NCHUNK // 4, body, 0)
    plsc.subcore_barrier()
    pltpu.sync_copy(acc.at[pl.ds(row0, RPT)], out_hbm.at[c, pl.ds(row0, RPT)])


@functools.cache
def _sc_kernels():
    mesh = plsc.VectorSubcoreMesh(
        core_axis_name="c", subcore_axis_name="s",
        num_cores=NC, num_subcores=NS)
    deg = pl.kernel(
        _deg_body,
        out_type=jax.ShapeDtypeStruct((NC, NPAD, D), jnp.float32),
        mesh=mesh,
        scratch_types=[
            pltpu.VMEM((2, CHUNK), jnp.int32),
            pltpu.VMEM((CHUNK, D), jnp.float32),
            pltpu.VMEM((CHUNK, D), jnp.float32),
            pltpu.VMEM_SHARED((NPAD, D), jnp.float32),
            pltpu.SemaphoreType.DMA,
        ])
    prop = pl.kernel(
        _prop_body,
        out_type=jax.ShapeDtypeStruct((NC, NPAD, D), jnp.float32),
        mesh=mesh,
        scratch_types=[
            pltpu.VMEM((4, 2, CHUNK), jnp.int32),
            pltpu.VMEM((CHUNK, D), jnp.float32),
            pltpu.VMEM((CHUNK, D), jnp.float32),
            pltpu.VMEM_SHARED((NPAD, D), jnp.float32),
            pltpu.SemaphoreType.DMA,
            pltpu.SemaphoreType.DMA,
            pltpu.SemaphoreType.DMA,
        ])
    return deg, prop


_BLK = 1000
_GRID = N // _BLK


def _din_dout(cnt):
    cs = sum(cnt[i] for i in range(NC))
    dout = jnp.maximum(cs[:, 0:1], 1.0)
    din = jnp.maximum(cs[:, 1:2], 1.0)
    return lax.rsqrt(din), lax.rsqrt(dout)


def _b1_body(x_ref, w_ref, cnt_ref, o_ref):
    _, dout = _din_dout(cnt_ref[...])
    o_ref[...] = jnp.dot(x_ref[...] * dout, w_ref[...],
                         preferred_element_type=jnp.float32)


def _mid_body(p_ref, cnt_ref, b_ref, w_ref, o_ref):
    din, dout = _din_dout(cnt_ref[...])
    x = sum(p_ref[i] for i in range(NC)) * din + b_ref[...]
    o_ref[...] = jnp.dot(x * dout, w_ref[...],
                         preferred_element_type=jnp.float32)


def _out_body(p_ref, cnt_ref, b_ref, g_ref, be_ref, o_ref):
    din, _ = _din_dout(cnt_ref[...])
    x = sum(p_ref[i] for i in range(NC)) * din + b_ref[...]
    mu = jnp.mean(x, axis=-1, keepdims=True)
    var = jnp.mean((x - mu) * (x - mu), axis=-1, keepdims=True)
    o_ref[...] = (x - mu) * lax.rsqrt(var + 1e-5) * g_ref[...] + be_ref[...]


_X_SPEC = pl.BlockSpec((_BLK, D), lambda i: (i, 0))
_W_SPEC = pl.BlockSpec((D, D), lambda i: (0, 0))
_CNT_SPEC = pl.BlockSpec((NC, _BLK, D), lambda i: (0, i, 0))
_P_SPEC = pl.BlockSpec((NC, _BLK, D), lambda i: (0, i, 0))
_V_SPEC = pl.BlockSpec((1, D), lambda i: (0, 0))
_O_SHAPE = jax.ShapeDtypeStruct((N, D), jnp.float32)

_b1_call = pl.pallas_call(
    _b1_body, grid=(_GRID,),
    in_specs=[_X_SPEC, _W_SPEC, _CNT_SPEC],
    out_specs=_X_SPEC, out_shape=_O_SHAPE)

_mid_call = pl.pallas_call(
    _mid_body, grid=(_GRID,),
    in_specs=[_P_SPEC, _CNT_SPEC, _V_SPEC, _W_SPEC],
    out_specs=_X_SPEC, out_shape=_O_SHAPE)

_out_call = pl.pallas_call(
    _out_body, grid=(_GRID,),
    in_specs=[_P_SPEC, _CNT_SPEC, _V_SPEC, _V_SPEC, _V_SPEC],
    out_specs=_X_SPEC, out_shape=_O_SHAPE)


def kernel(in_feat, edge_index, W1, b1, W2, b2, W3, b3, gamma, beta):
    src = edge_index[0].astype(jnp.int32)
    dst = edge_index[1].astype(jnp.int32)
    pad = E_PAD - E
    src_n = jnp.concatenate([src, jnp.full((pad,), N, jnp.int32)]).reshape(NW, NCHUNK, CHUNK)
    src_0 = jnp.concatenate([src, jnp.zeros((pad,), jnp.int32)]).reshape(NW, NCHUNK, CHUNK)
    dst_n = jnp.concatenate([dst, jnp.full((pad,), N, jnp.int32)]).reshape(NW, NCHUNK, CHUNK)
    e_deg = jnp.stack([src_n, dst_n], axis=2)
    e_prop = jnp.stack([src_0, dst_n], axis=2)
    eye = jnp.zeros((2, CHUNK, D), jnp.float32)
    eye = eye.at[0, :, 0].set(1.0).at[1, :, 1].set(1.0)
    zerosd = jnp.zeros((NPAD, D), jnp.float32)

    _deg_kernel, _prop_kernel = _sc_kernels()
    cnt = _deg_kernel(e_deg, eye, zerosd)
    t1 = _b1_call(in_feat, W1, cnt)
    p1 = _prop_kernel(t1, e_prop, zerosd)
    t2 = _mid_call(p1, cnt, b1.reshape(1, D), W2)
    p2 = _prop_kernel(t2, e_prop, zerosd)
    t3 = _mid_call(p2, cnt, b2.reshape(1, D), W3)
    p3 = _prop_kernel(t3, e_prop, zerosd)
    return _out_call(p3, cnt, b3.reshape(1, D), gamma.reshape(1, D),
                     beta.reshape(1, D))


# speedup vs baseline: 1.1914x; 1.1914x over previous
import functools

import jax
import jax.numpy as jnp
from jax import lax
from jax.experimental import pallas as pl
from jax.experimental.pallas import tpu as pltpu
from jax.experimental.pallas import tpu_sc as plsc

N = 10000
D = 128
E = 320000
NC, NS = 2, 16
NW = NC * NS
CHUNK = 128
NCHUNK = 80
EPT = NCHUNK * CHUNK
E_PAD = EPT * NW
NPAD = 10112
RPT = NPAD // NS


def _deg_body(edges_hbm, eye_hbm, zerosd_hbm, out_hbm,
              pair_v, e0_v, e1_v, acc, sem):
    c = lax.axis_index("c")
    s = lax.axis_index("s")
    wid = c * NS + s
    row0 = s * RPT
    pltpu.sync_copy(zerosd_hbm.at[pl.ds(row0, RPT)], acc.at[pl.ds(row0, RPT)])
    pltpu.sync_copy(eye_hbm.at[0], e0_v)
    pltpu.sync_copy(eye_hbm.at[1], e1_v)
    plsc.subcore_barrier()

    def body(i, carry):
        pltpu.async_copy(edges_hbm.at[wid, i], pair_v, sem).wait()
        pltpu.sync_copy(e0_v, acc.at[pair_v.at[0]], add=True)
        pltpu.sync_copy(e1_v, acc.at[pair_v.at[1]], add=True)
        return carry

    lax.fori_loop(0, NCHUNK, body, 0)
    plsc.subcore_barrier()
    pltpu.sync_copy(acc.at[pl.ds(row0, RPT)], out_hbm.at[c, pl.ds(row0, RPT)])


def _prop_body(t_hbm, edges_hbm, zerosd_hbm, out_hbm,
               idx_v, rows0_v, rows1_v, acc, sem_i, sem_g0, sem_g1,
               sem_s0, sem_s1):
    c = lax.axis_index("c")
    s = lax.axis_index("s")
    wid = c * NS + s
    row0 = s * RPT
    pltpu.sync_copy(zerosd_hbm.at[pl.ds(row0, RPT)], acc.at[pl.ds(row0, RPT)])
    plsc.subcore_barrier()

    rows_v = (rows0_v, rows1_v)
    sem_g = (sem_g0, sem_g1)
    sem_s = (sem_s0, sem_s1)

    def wait_idx():
        pltpu.make_async_copy(edges_hbm.at[wid, 0], idx_v.at[0], sem_i).wait()

    def wait_rows(b, sems):
        pltpu.make_async_copy(t_hbm.at[idx_v.at[0, 0]], rows_v[b], sems[b]).wait()

    pltpu.async_copy(edges_hbm.at[wid, 0], idx_v.at[0], sem_i)
    pltpu.async_copy(edges_hbm.at[wid, 1], idx_v.at[1], sem_i)
    pltpu.async_copy(edges_hbm.at[wid, 2], idx_v.at[2], sem_i)
    wait_idx()
    pltpu.async_copy(t_hbm.at[idx_v.at[0, 0]], rows0_v, sem_g0)

    def body(g, carry):
        for p in range(4):
            i = 4 * g + p

            @pl.when(i >= 1)
            def _():
                wait_rows((p + 1) % 2, sem_s)

            @pl.when(i + 1 < NCHUNK)
            def _():
                wait_idx()
                pltpu.async_copy(t_hbm.at[idx_v.at[(p + 1) % 4, 0]],
                                 rows_v[(p + 1) % 2], sem_g[(p + 1) % 2])

            @pl.when(i + 3 < NCHUNK)
            def _():
                pltpu.async_copy(edges_hbm.at[wid, i + 3],
                                 idx_v.at[(p + 3) % 4], sem_i)

            wait_rows(p % 2, sem_g)
            pltpu.async_copy(rows_v[p % 2], acc.at[idx_v.at[p, 1]],
                             sem_s[p % 2], add=True)
        return carry

    lax.fori_loop(0, NCHUNK // 4, body, 0)
    wait_rows((NCHUNK - 1) % 2, sem_s)
    plsc.subcore_barrier()
    pltpu.sync_copy(acc.at[pl.ds(row0, RPT)], out_hbm.at[c, pl.ds(row0, RPT)])


@functools.cache
def _sc_kernels():
    mesh = plsc.VectorSubcoreMesh(
        core_axis_name="c", subcore_axis_name="s",
        num_cores=NC, num_subcores=NS)
    deg = pl.kernel(
        _deg_body,
        out_type=jax.ShapeDtypeStruct((NC, NPAD, D), jnp.float32),
        mesh=mesh,
        scratch_types=[
            pltpu.VMEM((2, CHUNK), jnp.int32),
            pltpu.VMEM((CHUNK, D), jnp.float32),
            pltpu.VMEM((CHUNK, D), jnp.float32),
            pltpu.VMEM_SHARED((NPAD, D), jnp.float32),
            pltpu.SemaphoreType.DMA,
        ])
    prop = pl.kernel(
        _prop_body,
        out_type=jax.ShapeDtypeStruct((NC, NPAD, D), jnp.float32),
        mesh=mesh,
        scratch_types=[
            pltpu.VMEM((4, 2, CHUNK), jnp.int32),
            pltpu.VMEM((CHUNK, D), jnp.float32),
            pltpu.VMEM((CHUNK, D), jnp.float32),
            pltpu.VMEM_SHARED((NPAD, D), jnp.float32),
            pltpu.SemaphoreType.DMA,
            pltpu.SemaphoreType.DMA,
            pltpu.SemaphoreType.DMA,
            pltpu.SemaphoreType.DMA,
            pltpu.SemaphoreType.DMA,
        ])
    return deg, prop


_BLK = 1000
_GRID = N // _BLK


def _din_dout(cnt):
    cs = sum(cnt[i] for i in range(NC))
    dout = jnp.maximum(cs[:, 0:1], 1.0)
    din = jnp.maximum(cs[:, 1:2], 1.0)
    return lax.rsqrt(din), lax.rsqrt(dout)


def _b1_body(x_ref, w_ref, cnt_ref, o_ref):
    _, dout = _din_dout(cnt_ref[...])
    o_ref[...] = jnp.dot(x_ref[...] * dout, w_ref[...],
                         preferred_element_type=jnp.float32)


def _mid_body(p_ref, cnt_ref, b_ref, w_ref, o_ref):
    din, dout = _din_dout(cnt_ref[...])
    x = sum(p_ref[i] for i in range(NC)) * din + b_ref[...]
    o_ref[...] = jnp.dot(x * dout, w_ref[...],
                         preferred_element_type=jnp.float32)


def _out_body(p_ref, cnt_ref, b_ref, g_ref, be_ref, o_ref):
    din, _ = _din_dout(cnt_ref[...])
    x = sum(p_ref[i] for i in range(NC)) * din + b_ref[...]
    mu = jnp.mean(x, axis=-1, keepdims=True)
    var = jnp.mean((x - mu) * (x - mu), axis=-1, keepdims=True)
    o_ref[...] = (x - mu) * lax.rsqrt(var + 1e-5) * g_ref[...] + be_ref[...]


_X_SPEC = pl.BlockSpec((_BLK, D), lambda i: (i, 0))
_W_SPEC = pl.BlockSpec((D, D), lambda i: (0, 0))
_CNT_SPEC = pl.BlockSpec((NC, _BLK, D), lambda i: (0, i, 0))
_P_SPEC = pl.BlockSpec((NC, _BLK, D), lambda i: (0, i, 0))
_V_SPEC = pl.BlockSpec((1, D), lambda i: (0, 0))
_O_SHAPE = jax.ShapeDtypeStruct((N, D), jnp.float32)

_b1_call = pl.pallas_call(
    _b1_body, grid=(_GRID,),
    in_specs=[_X_SPEC, _W_SPEC, _CNT_SPEC],
    out_specs=_X_SPEC, out_shape=_O_SHAPE)

_mid_call = pl.pallas_call(
    _mid_body, grid=(_GRID,),
    in_specs=[_P_SPEC, _CNT_SPEC, _V_SPEC, _W_SPEC],
    out_specs=_X_SPEC, out_shape=_O_SHAPE)

_out_call = pl.pallas_call(
    _out_body, grid=(_GRID,),
    in_specs=[_P_SPEC, _CNT_SPEC, _V_SPEC, _V_SPEC, _V_SPEC],
    out_specs=_X_SPEC, out_shape=_O_SHAPE)


def kernel(in_feat, edge_index, W1, b1, W2, b2, W3, b3, gamma, beta):
    src = edge_index[0].astype(jnp.int32)
    dst = edge_index[1].astype(jnp.int32)
    pad = E_PAD - E
    src_n = jnp.concatenate([src, jnp.full((pad,), N, jnp.int32)]).reshape(NW, NCHUNK, CHUNK)
    src_0 = jnp.concatenate([src, jnp.zeros((pad,), jnp.int32)]).reshape(NW, NCHUNK, CHUNK)
    dst_n = jnp.concatenate([dst, jnp.full((pad,), N, jnp.int32)]).reshape(NW, NCHUNK, CHUNK)
    e_deg = jnp.stack([src_n, dst_n], axis=2)
    e_prop = jnp.stack([src_0, dst_n], axis=2)
    eye = jnp.zeros((2, CHUNK, D), jnp.float32)
    eye = eye.at[0, :, 0].set(1.0).at[1, :, 1].set(1.0)
    zerosd = jnp.zeros((NPAD, D), jnp.float32)

    _deg_kernel, _prop_kernel = _sc_kernels()
    cnt = _deg_kernel(e_deg, eye, zerosd)
    t1 = _b1_call(in_feat, W1, cnt)
    p1 = _prop_kernel(t1, e_prop, zerosd)
    t2 = _mid_call(p1, cnt, b1.reshape(1, D), W2)
    p2 = _prop_kernel(t2, e_prop, zerosd)
    t3 = _mid_call(p2, cnt, b2.reshape(1, D), W3)
    p3 = _prop_kernel(t3, e_prop, zerosd)
    return _out_call(p3, cnt, b3.reshape(1, D), gamma.reshape(1, D),
                     beta.reshape(1, D))


# speedup vs baseline: 1.5126x; 1.2696x over previous
import functools

import jax
import jax.numpy as jnp
from jax import lax
from jax.experimental import pallas as pl
from jax.experimental.pallas import tpu as pltpu
from jax.experimental.pallas import tpu_sc as plsc

N = 10000
D = 128
E = 320000
NC, NS = 2, 16
NW = NC * NS
CHUNK = 128
NCHUNK = 79
EPT = NCHUNK * CHUNK
E_PAD = EPT * NW
NPAD = 10112
RPT = NPAD // NS


def _deg_body(edges_hbm, eye_hbm, zerosd_hbm, out_hbm,
              pair_v, e0_v, e1_v, acc, sem):
    c = lax.axis_index("c")
    s = lax.axis_index("s")
    wid = c * NS + s
    row0 = s * RPT
    pltpu.sync_copy(zerosd_hbm.at[pl.ds(row0, RPT)], acc.at[pl.ds(row0, RPT)])
    pltpu.sync_copy(eye_hbm.at[0], e0_v)
    pltpu.sync_copy(eye_hbm.at[1], e1_v)
    plsc.subcore_barrier()

    def body(i, carry):
        pltpu.async_copy(edges_hbm.at[wid, i], pair_v, sem).wait()
        pltpu.sync_copy(e0_v, acc.at[pair_v.at[0]], add=True)
        pltpu.sync_copy(e1_v, acc.at[pair_v.at[1]], add=True)
        return carry

    lax.fori_loop(0, NCHUNK, body, 0)
    plsc.subcore_barrier()
    pltpu.sync_copy(acc.at[pl.ds(row0, RPT)], out_hbm.at[c, pl.ds(row0, RPT)])


def _prop_body(t_hbm, src0_hbm, dst_hbm, zerosd_hbm, out_hbm,
               sidx_v, didx_v, rows_v, acc, sem):
    c = lax.axis_index("c")
    s = lax.axis_index("s")
    wid = c * NS + s
    row0 = s * RPT
    pltpu.sync_copy(zerosd_hbm.at[pl.ds(row0, RPT)], acc.at[pl.ds(row0, RPT)])
    pltpu.sync_copy(src0_hbm.at[wid], sidx_v)
    pltpu.sync_copy(dst_hbm.at[wid], didx_v)
    plsc.subcore_barrier()

    def body(i, carry):
        pltpu.async_copy(t_hbm.at[sidx_v.at[i]], rows_v, sem).wait()
        pltpu.sync_copy(rows_v, acc.at[didx_v.at[i]], add=True)
        return carry

    lax.fori_loop(0, NCHUNK, body, 0)
    plsc.subcore_barrier()
    pltpu.sync_copy(acc.at[pl.ds(row0, RPT)], out_hbm.at[c, pl.ds(row0, RPT)])


@functools.cache
def _sc_kernels():
    mesh = plsc.VectorSubcoreMesh(
        core_axis_name="c", subcore_axis_name="s",
        num_cores=NC, num_subcores=NS)
    deg = pl.kernel(
        _deg_body,
        out_type=jax.ShapeDtypeStruct((NC, NPAD, D), jnp.float32),
        mesh=mesh,
        scratch_types=[
            pltpu.VMEM((2, CHUNK), jnp.int32),
            pltpu.VMEM((CHUNK, D), jnp.float32),
            pltpu.VMEM((CHUNK, D), jnp.float32),
            pltpu.VMEM_SHARED((NPAD, D), jnp.float32),
            pltpu.SemaphoreType.DMA,
        ])
    prop = pl.kernel(
        _prop_body,
        out_type=jax.ShapeDtypeStruct((NC, NPAD, D), jnp.float32),
        mesh=mesh,
        scratch_types=[
            pltpu.VMEM((NCHUNK, CHUNK), jnp.int32),
            pltpu.VMEM((NCHUNK, CHUNK), jnp.int32),
            pltpu.VMEM((CHUNK, D), jnp.float32),
            pltpu.VMEM_SHARED((NPAD, D), jnp.float32),
            pltpu.SemaphoreType.DMA,
        ])
    return deg, prop


_BLK = 1000
_GRID = N // _BLK


def _din_dout(cnt):
    cs = sum(cnt[i] for i in range(NC))
    dout = jnp.maximum(cs[:, 0:1], 1.0)
    din = jnp.maximum(cs[:, 1:2], 1.0)
    return lax.rsqrt(din), lax.rsqrt(dout)


def _b1_body(x_ref, w_ref, cnt_ref, o_ref):
    _, dout = _din_dout(cnt_ref[...])
    o_ref[...] = jnp.dot(x_ref[...] * dout, w_ref[...],
                         preferred_element_type=jnp.float32)


def _mid_body(p_ref, cnt_ref, b_ref, w_ref, o_ref):
    din, dout = _din_dout(cnt_ref[...])
    x = sum(p_ref[i] for i in range(NC)) * din + b_ref[...]
    o_ref[...] = jnp.dot(x * dout, w_ref[...],
                         preferred_element_type=jnp.float32)


def _out_body(p_ref, cnt_ref, b_ref, g_ref, be_ref, o_ref):
    din, _ = _din_dout(cnt_ref[...])
    x = sum(p_ref[i] for i in range(NC)) * din + b_ref[...]
    mu = jnp.mean(x, axis=-1, keepdims=True)
    var = jnp.mean((x - mu) * (x - mu), axis=-1, keepdims=True)
    o_ref[...] = (x - mu) * lax.rsqrt(var + 1e-5) * g_ref[...] + be_ref[...]


_X_SPEC = pl.BlockSpec((_BLK, D), lambda i: (i, 0))
_W_SPEC = pl.BlockSpec((D, D), lambda i: (0, 0))
_CNT_SPEC = pl.BlockSpec((NC, _BLK, D), lambda i: (0, i, 0))
_P_SPEC = pl.BlockSpec((NC, _BLK, D), lambda i: (0, i, 0))
_V_SPEC = pl.BlockSpec((1, D), lambda i: (0, 0))
_O_SHAPE = jax.ShapeDtypeStruct((N, D), jnp.float32)

_b1_call = pl.pallas_call(
    _b1_body, grid=(_GRID,),
    in_specs=[_X_SPEC, _W_SPEC, _CNT_SPEC],
    out_specs=_X_SPEC, out_shape=_O_SHAPE)

_mid_call = pl.pallas_call(
    _mid_body, grid=(_GRID,),
    in_specs=[_P_SPEC, _CNT_SPEC, _V_SPEC, _W_SPEC],
    out_specs=_X_SPEC, out_shape=_O_SHAPE)

_out_call = pl.pallas_call(
    _out_body, grid=(_GRID,),
    in_specs=[_P_SPEC, _CNT_SPEC, _V_SPEC, _V_SPEC, _V_SPEC],
    out_specs=_X_SPEC, out_shape=_O_SHAPE)


def kernel(in_feat, edge_index, W1, b1, W2, b2, W3, b3, gamma, beta):
    src = edge_index[0].astype(jnp.int32)
    dst = edge_index[1].astype(jnp.int32)
    pad = E_PAD - E
    src_n = jnp.concatenate([src, jnp.full((pad,), N, jnp.int32)]).reshape(NW, NCHUNK, CHUNK)
    src_0 = jnp.concatenate([src, jnp.zeros((pad,), jnp.int32)]).reshape(NW, NCHUNK, CHUNK)
    dst_n = jnp.concatenate([dst, jnp.full((pad,), N, jnp.int32)]).reshape(NW, NCHUNK, CHUNK)
    e_deg = jnp.stack([src_n, dst_n], axis=2)
    eye = jnp.zeros((2, CHUNK, D), jnp.float32)
    eye = eye.at[0, :, 0].set(1.0).at[1, :, 1].set(1.0)
    zerosd = jnp.zeros((NPAD, D), jnp.float32)

    _deg_kernel, _prop_kernel = _sc_kernels()
    cnt = _deg_kernel(e_deg, eye, zerosd)
    t1 = _b1_call(in_feat, W1, cnt)
    p1 = _prop_kernel(t1, src_0, dst_n, zerosd)
    t2 = _mid_call(p1, cnt, b1.reshape(1, D), W2)
    p2 = _prop_kernel(t2, src_0, dst_n, zerosd)
    t3 = _mid_call(p2, cnt, b2.reshape(1, D), W3)
    p3 = _prop_kernel(t3, src_0, dst_n, zerosd)
    return _out_call(p3, cnt, b3.reshape(1, D), gamma.reshape(1, D),
                     beta.reshape(1, D))
